# Initial kernel scaffold; baseline (speedup 1.0000x reference)
import jax
import jax.numpy as jnp
from jax.experimental import pallas as pl


def kernel(reg, batch_idx, row_idx, col_idx):
    raise NotImplementedError("write your pallas kernel here")


# speedup vs baseline: 4.3940x; 4.3940x over previous
import functools

import jax
import jax.numpy as jnp
from jax import lax
from jax.experimental import pallas as pl
from jax.experimental.pallas import tpu as pltpu
from jax.experimental.pallas import tpu_sc as plsc

N = 2_000_000


def _build(NB=4, GS=512, NW=32, PTS_W=65_536, WB=2048, WC=1024, CHUNK=128,
           interpret=False):
    N_P = NW * PTS_W
    CPB = (NB * GS * GS) // NW
    SH = CPB.bit_length() - 1
    SHA = SH - (GS.bit_length() - 1)
    N_ALLOC = N_P + 8 * NW + WC
    NLANE = 16
    NBIN = NW + 1

    mesh = plsc.VectorSubcoreMesh(core_axis_name="c", subcore_axis_name="s",
                                  num_cores=2, num_subcores=16)
    params = pltpu.CompilerParams(needs_layout_passes=False,
                                  use_tc_tiling_on_sc=False)

    def _wid():
        return lax.axis_index("s") * 2 + lax.axis_index("c")

    iota = lambda: lax.iota(jnp.int32, 16)

    @functools.partial(
        pl.kernel,
        out_type=jax.ShapeDtypeStruct((NW, NLANE * NBIN), jnp.int32),
        mesh=mesh,
        compiler_params=params,
        interpret=interpret,
        scratch_types=[
            pltpu.VMEM((NLANE * NBIN,), jnp.int32),
            pltpu.VMEM((WB,), jnp.int32),
            pltpu.VMEM((WB,), jnp.int32),
        ],
    )
    def phase_a(b_hbm, r_hbm, counts_hbm, cnt_v, bwin, rwin):
        wid = _wid()
        io = iota()
        zero16 = jnp.zeros((16,), jnp.int32)

        def z(k, carry):
            cnt_v[pl.ds(k * 16, 16)] = zero16
            return carry

        lax.fori_loop(0, NBIN, z, 0)

        def win(g, carry):
            base = wid * PTS_W + g * WB
            pltpu.sync_copy(b_hbm.at[pl.ds(base, WB)], bwin)
            pltpu.sync_copy(r_hbm.at[pl.ds(base, WB)], rwin)

            def vr(k, c2):
                bv = bwin[pl.ds(k * 16, 16)]
                rv = rwin[pl.ds(k * 16, 16)]
                binv = (bv * GS + rv) >> SHA
                pidx = io * NBIN + binv
                cur = plsc.load_gather(cnt_v, [pidx])
                plsc.store_scatter(cnt_v, [pidx], cur + 1)
                return c2

            return lax.fori_loop(0, WB // 16, vr, carry)

        lax.fori_loop(0, PTS_W // WB, win, 0)
        pltpu.sync_copy(cnt_v, counts_hbm.at[wid])

    @functools.partial(
        pl.kernel,
        out_type=jax.ShapeDtypeStruct((N_ALLOC, 16), jnp.int32),
        mesh=mesh,
        compiler_params=params,
        interpret=interpret,
        scratch_types=[
            pltpu.VMEM((NLANE * NBIN,), jnp.int32),
            pltpu.VMEM((WB,), jnp.int32),
            pltpu.VMEM((WB,), jnp.int32),
            pltpu.VMEM((WB,), jnp.int32),
            pltpu.VMEM((WB, 2), jnp.float32),
            pltpu.VMEM((WB, 16), jnp.int32),
            pltpu.VMEM((WB // CHUNK, CHUNK), jnp.int32),
            pltpu.SemaphoreType.DMA,
        ],
    )
    def phase_b(b_hbm, r_hbm, c_hbm, reg_hbm, bases_hbm, binned_hbm,
                ptr_v, bwin, rwin, cwin, regwin, recbuf, destb, sem):
        wid = _wid()
        io = iota()
        col0 = jnp.zeros((16,), jnp.int32)
        col1 = col0 + 1
        col2 = col0 + 2
        col3 = col0 + 3

        pltpu.sync_copy(bases_hbm.at[wid], ptr_v)

        def win(g, carry):
            base = wid * PTS_W + g * WB
            pltpu.sync_copy(b_hbm.at[pl.ds(base, WB)], bwin)
            pltpu.sync_copy(r_hbm.at[pl.ds(base, WB)], rwin)
            pltpu.sync_copy(c_hbm.at[pl.ds(base, WB)], cwin)
            pltpu.sync_copy(reg_hbm.at[pl.ds(base, WB)], regwin)

            def vr(k, c2):
                loc = k * 16 + io
                bv = bwin[pl.ds(k * 16, 16)]
                rv = rwin[pl.ds(k * 16, 16)]
                cv = cwin[pl.ds(k * 16, 16)]
                cell = (bv * GS + rv) * GS + cv
                binv = cell >> SH
                lcell = cell & (CPB - 1)
                pidx = io * NBIN + binv
                pv = plsc.load_gather(ptr_v, [pidx])
                plsc.store_scatter(ptr_v, [pidx], pv + 1)
                iv = base + k * 16 + io
                r0 = plsc.bitcast(plsc.load_gather(regwin, [loc, col0]),
                                  jnp.int32)
                r1 = plsc.bitcast(plsc.load_gather(regwin, [loc, col1]),
                                  jnp.int32)
                plsc.store_scatter(recbuf, [loc, col0], lcell)
                plsc.store_scatter(recbuf, [loc, col1], iv)
                plsc.store_scatter(recbuf, [loc, col2], r0)
                plsc.store_scatter(recbuf, [loc, col3], r1)
                rowv = col0 + (k // (CHUNK // 16))
                cpos = (k % (CHUNK // 16)) * 16 + io
                plsc.store_scatter(destb, [rowv, cpos], pv)
                return c2

            lax.fori_loop(0, WB // 16, vr, 0)

            def chs(ch, c3):
                pltpu.async_copy(recbuf.at[pl.ds(ch * CHUNK, CHUNK), :],
                                 binned_hbm.at[destb.at[ch]], sem).wait()
                return c3

            return lax.fori_loop(0, WB // CHUNK, chs, carry)

        lax.fori_loop(0, PTS_W // WB, win, 0)

    @functools.partial(
        pl.kernel,
        out_type=jax.ShapeDtypeStruct((NB * GS * GS * 2,), jnp.float32),
        mesh=mesh,
        compiler_params=params,
        interpret=interpret,
        scratch_types=[
            pltpu.VMEM((CPB,), jnp.int32),
            pltpu.VMEM((2 * CPB,), jnp.float32),
            pltpu.VMEM((WC, 16), jnp.int32),
            pltpu.VMEM((2, 16), jnp.int32),
            pltpu.VMEM((2 * WC,), jnp.float32),
        ],
    )
    def phase_c(binned_hbm, seg_hbm, conf_hbm, win_v, ev, recwin, segw,
                outw):
        wid = _wid()
        io = iota()
        zero16f = jnp.zeros((16,), jnp.float32)
        neg1 = jnp.zeros((16,), jnp.int32) - 1

        pltpu.sync_copy(seg_hbm.at[wid], segw)
        start = jnp.max(segw[0, :])
        cnt = jnp.max(segw[1, :])
        col0 = jnp.zeros((16,), jnp.int32)
        col1 = col0 + 1
        col2 = col0 + 2
        col3 = col0 + 3

        def zw(k, carry):
            win_v[pl.ds(k * 16, 16)] = neg1
            return carry

        lax.fori_loop(0, CPB // 16, zw, 0)

        def ze(k, carry):
            ev[pl.ds(k * 16, 16)] = zero16f
            return carry

        lax.fori_loop(0, 2 * CPB // 16, ze, 0)

        nwin = (cnt + WC - 1) // WC

        def c1(g, carry):
            pltpu.sync_copy(binned_hbm.at[pl.ds(start + g * WC, WC)],
                            recwin)

            def vr(k, c2):
                loc = k * 16 + io
                valid = (g * WC + k * 16 + io) < cnt
                cellv = plsc.load_gather(recwin, [loc, col0]) & (CPB - 1)
                iv = plsc.load_gather(recwin, [loc, col1])
                ivs = jnp.where(valid, iv, -1)
                cur = plsc.load_gather(win_v, [cellv])

                def cond(cu):
                    return jnp.any(ivs > cu)

                def body(cu):
                    plsc.store_scatter(win_v, [cellv], ivs, mask=ivs > cu)
                    return plsc.load_gather(win_v, [cellv])

                lax.while_loop(cond, body, cur)
                return c2

            return lax.fori_loop(0, WC // 16, vr, carry)

        lax.fori_loop(0, nwin, c1, 0)

        def c2p(g, carry):
            pltpu.sync_copy(binned_hbm.at[pl.ds(start + g * WC, WC)],
                            recwin)

            def vr(k, c2):
                loc = k * 16 + io
                valid = (g * WC + k * 16 + io) < cnt
                cellv = plsc.load_gather(recwin, [loc, col0]) & (CPB - 1)
                iv = plsc.load_gather(recwin, [loc, col1])
                ivs = jnp.where(valid, iv, -1)
                r0 = plsc.bitcast(plsc.load_gather(recwin, [loc, col2]),
                                  jnp.float32)
                r1 = plsc.bitcast(plsc.load_gather(recwin, [loc, col3]),
                                  jnp.float32)
                w = plsc.load_gather(win_v, [cellv])
                mok = (ivs == w) & valid
                plsc.store_scatter(ev, [cellv * 2], r0, mask=mok)
                plsc.store_scatter(ev, [cellv * 2 + 1], r1, mask=mok)
                return c2

            return lax.fori_loop(0, WC // 16, vr, carry)

        lax.fori_loop(0, nwin, c2p, 0)

        def fw(g, carry):
            def vr(k, c2):
                cc = (g * WC + k * 16 + io) * 2
                v0 = plsc.load_gather(ev, [cc])
                v1 = plsc.load_gather(ev, [cc + 1])
                m = jnp.maximum(v0, v1)
                e0 = jnp.exp(v0 - m)
                e1 = jnp.exp(v1 - m)
                s = e0 + e1
                lo = (k * 16 + io) * 2
                plsc.store_scatter(outw, [lo], e0 / s)
                plsc.store_scatter(outw, [lo + 1], e1 / s)
                return c2

            lax.fori_loop(0, WC // 16, vr, 0)
            pltpu.sync_copy(
                outw,
                conf_hbm.at[pl.ds(wid * 2 * CPB + g * 2 * WC, 2 * WC)])
            return carry

        lax.fori_loop(0, CPB // WC, fw, 0)

    def run(reg, batch_idx, row_idx, col_idx, n, debug=False):
        pad = N_P - n
        i32 = jnp.int32
        b_p = jnp.concatenate([batch_idx.astype(i32),
                               jnp.full((pad,), NB, i32)])
        r_p = jnp.concatenate([row_idx.astype(i32),
                               jnp.zeros((pad,), i32)])
        c_p = jnp.concatenate([col_idx.astype(i32),
                               jnp.zeros((pad,), i32)])
        reg_p = jnp.concatenate([reg, jnp.zeros((pad, 2), reg.dtype)])

        counts = phase_a(b_p, r_p)
        cnts = counts.reshape(NW, NLANE, NBIN)
        per_bin = cnts.transpose(2, 0, 1).reshape(NBIN, NW * NLANE)
        tot = per_bin.sum(axis=1)
        tot_pad = ((tot + 7) // 8) * 8
        segstart = jnp.concatenate(
            [jnp.zeros((1,), i32),
             jnp.cumsum(tot_pad).astype(i32)])[:NBIN]
        within = jnp.cumsum(per_bin, axis=1).astype(i32) - per_bin
        bases = (segstart[:, None] + within).reshape(NBIN, NW, NLANE)
        bases = bases.transpose(1, 2, 0).reshape(NW, NLANE * NBIN)

        binned = phase_b(b_p, r_p, c_p, reg_p, bases)

        seg = jnp.stack([segstart[:NW], tot[:NW].astype(i32)], axis=1)
        seg = jnp.broadcast_to(seg[:, :, None], (NW, 2, 16)).astype(i32)
        conf_flat = phase_c(binned, seg)
        if debug:
            return dict(counts=counts, bases=bases, binned=binned, seg=seg,
                        conf=conf_flat.reshape(NB, GS, GS, 2))
        return conf_flat.reshape(NB, GS, GS, 2)

    return run


_run = _build()


def kernel(reg, batch_idx, row_idx, col_idx):
    return _run(reg, batch_idx, row_idx, col_idx, N)
